# baseline (speedup 1.0000x reference)
import functools

import jax
import jax.numpy as jnp
from jax import lax
from jax.experimental import pallas as pl
from jax.experimental.pallas import tpu as pltpu
from jax.experimental.pallas import tpu_sc as plsc

_VOCAB = 1000000
_DIM = 32
_BATCH = 4096
_HIST = 50

_NC = 2
_NS = 16
_NW = _NC * _NS
_NBT = _BATCH // 128
_UNITS = _HIST * _NBT
_UPW = _UNITS // _NW

_NJ = _VOCAB // 128
_JPW = _NJ // _NW
_NJR = _NJ - _JPW * _NW
_VPAD = 1000064

_mesh = plsc.VectorSubcoreMesh(core_axis_name="c", subcore_axis_name="s")


@functools.partial(
    pl.kernel,
    out_type=jax.ShapeDtypeStruct((_VPAD * 2 * _DIM,), jnp.float32),
    mesh=_mesh,
    scratch_types=[
        pltpu.VMEM((_DIM, 128), jnp.float32),
        pltpu.VMEM((_DIM, 128), jnp.float32),
        pltpu.VMEM((128 * 2 * _DIM,), jnp.float32),
        pltpu.SemaphoreType.DMA,
    ],
    compiler_params=pltpu.CompilerParams(
        use_tc_tiling_on_sc=True, needs_layout_passes=False),
)
def _format(tr_hbm, ti_hbm, ct_hbm, buf_r, buf_i, ct_buf, sem):
    wid = lax.axis_index("s") * _NC + lax.axis_index("c")
    iota16 = lax.iota(jnp.int32, 16)
    nj = _JPW + jnp.where(wid < _NJR + 1, 1, 0)

    def block(k, carry):
        j = jnp.where(k < _JPW, wid * _JPW + k, _JPW * _NW + wid)
        partial = j >= _NJ
        v0 = pl.multiple_of(jnp.minimum(j, _NJ) * 128, 128)
        cp_r = pltpu.async_copy(tr_hbm.at[:, pl.ds(v0, 128)], buf_r, sem)
        cp_i = pltpu.async_copy(ti_hbm.at[:, pl.ds(v0, 128)], buf_i, sem)
        cp_r.wait()
        cp_i.wait()

        def tcol(c, tc):
            base = c * (2 * _DIM)
            cvec = jnp.full((16,), c, dtype=jnp.int32)
            for dhalf in range(_DIM // 16):
                dlanes = iota16 + (16 * dhalf)
                vr = plsc.load_gather(buf_r, [dlanes, cvec])
                vi = plsc.load_gather(buf_i, [dlanes, cvec])
                ct_buf[pl.ds(base + 16 * dhalf, 16)] = vr
                ct_buf[pl.ds(base + 16 * dhalf + _DIM, 16)] = vi
            return tc

        lax.fori_loop(0, 128, tcol, 0)
        half = 64 * 2 * _DIM
        dst0 = v0 * 2 * _DIM
        pltpu.sync_copy(ct_buf.at[pl.ds(0, half)],
                        ct_hbm.at[pl.ds(dst0, half)])

        @pl.when(jnp.logical_not(partial))
        def _():
            pltpu.sync_copy(ct_buf.at[pl.ds(half, half)],
                            ct_hbm.at[pl.ds(dst0 + half, half)])
        return carry

    lax.fori_loop(0, nj, block, 0)


@functools.partial(
    pl.kernel,
    out_type=(
        jax.ShapeDtypeStruct((_HIST, _DIM // 8, _NBT, 8, 128), jnp.float32),
        jax.ShapeDtypeStruct((_HIST, _DIM // 8, _NBT, 8, 128), jnp.float32),
    ),
    mesh=_mesh,
    scratch_types=[
        pltpu.VMEM((128,), jnp.int32),
        pltpu.VMEM((128, 2 * _DIM), jnp.float32),
        pltpu.VMEM((_DIM, 128), jnp.float32),
        pltpu.VMEM((_DIM, 128), jnp.float32),
        pltpu.SemaphoreType.DMA,
        pltpu.SemaphoreType.DMA,
    ],
    compiler_params=pltpu.CompilerParams(
        use_tc_tiling_on_sc=False, needs_layout_passes=False),
)
def _gather(ids_hbm, ct_hbm, out_r_hbm, out_i_hbm,
            idx_v, rows, out_tr, out_ti, sem_g, sem_o):
    wid = lax.axis_index("s") * _NC + lax.axis_index("c")
    ubase = wid * _UPW
    iota16 = lax.iota(jnp.int32, 16)

    def unit(k, carry):
        u = ubase + k
        h = u // _NBT
        bt = u % _NBT
        pltpu.sync_copy(ids_hbm.at[h, pl.ds(bt * 128, 128)], idx_v)
        pltpu.async_copy(ct_hbm.at[idx_v], rows, sem_g).wait()

        def trow(c, tc):
            cvec = jnp.full((16,), c, dtype=jnp.int32)
            for dhalf in range(_DIM // 16):
                dlanes = iota16 + (16 * dhalf)
                vr = rows[c, pl.ds(16 * dhalf, 16)]
                vi = rows[c, pl.ds(16 * dhalf + _DIM, 16)]
                plsc.store_scatter(out_tr, [dlanes, cvec], vr)
                plsc.store_scatter(out_ti, [dlanes, cvec], vi)
            return tc

        lax.fori_loop(0, 128, trow, 0)
        ocps = []
        for dt in range(_DIM // 8):
            ocps.append(pltpu.async_copy(
                out_tr.at[pl.ds(dt * 8, 8)], out_r_hbm.at[h, dt, bt], sem_o))
            ocps.append(pltpu.async_copy(
                out_ti.at[pl.ds(dt * 8, 8)], out_i_hbm.at[h, dt, bt], sem_o))
        for cp in ocps:
            cp.wait()
        return carry

    lax.fori_loop(0, _UPW, unit, 0)


def kernel(input_ids, emb_real, emb_imag):
    ids_t = input_ids.T.astype(jnp.int32)
    ct_flat = _format(emb_real.T, emb_imag.T)
    ct = ct_flat.reshape(_VPAD, 2 * _DIM)
    o_r, o_i = _gather(ids_t, ct)
    p_r = o_r.transpose(0, 1, 3, 2, 4).reshape(_HIST, _DIM, _BATCH)
    p_i = o_i.transpose(0, 1, 3, 2, 4).reshape(_HIST, _DIM, _BATCH)
    out_t = lax.complex(p_r, p_i)
    return out_t.transpose(2, 0, 1)
